# baseline (speedup 1.0000x reference)
import functools

import jax
import jax.numpy as jnp
from jax import lax
from jax.experimental import pallas as pl
from jax.experimental.pallas import tpu as pltpu
from jax.experimental.pallas import tpu_sc as plsc

NC = 2
NS = 16
NW = NC * NS

D = 32
NB = 16384
SEQ = 50
B_TOTAL = NB * SEQ
B_PER_W = B_TOTAL // NW
NB_PER_W = NB // NW
BGRP = 16
CHUNK = BGRP * SEQ
N_GROUPS = NB_PER_W // BGRP
NBUF = 2


def _make_kernel():
  mesh = plsc.VectorSubcoreMesh(core_axis_name="c", subcore_axis_name="s")

  @functools.partial(
      pl.kernel,
      mesh=mesh,
      out_type=jax.ShapeDtypeStruct((SEQ, D // 8, NB // 128, 8, 128),
                                    jnp.float32),
      scratch_types=[
          pltpu.VMEM((B_PER_W,), jnp.int32),
          *[pltpu.VMEM((CHUNK, D), jnp.float32) for _ in range(NBUF)],
          *[pltpu.VMEM((SEQ, D // 8, 8, BGRP), jnp.float32)
            for _ in range(NBUF)],
          *[pltpu.SemaphoreType.DMA for _ in range(2 * NBUF)],
      ],
      compiler_params=pltpu.CompilerParams(
          use_tc_tiling_on_sc=False, needs_layout_passes=False
      ),
  )
  def gather_kernel(idx_hbm, table_hbm, out_hbm, idx_all, *bufs_and_sems):
    rows = bufs_and_sems[:NBUF]
    trs = bufs_and_sems[NBUF:2 * NBUF]
    sem_g = bufs_and_sems[2 * NBUF:3 * NBUF]
    sem_o = bufs_and_sems[3 * NBUF:]

    wid = lax.axis_index("s") * NC + lax.axis_index("c")
    base_w = wid * B_PER_W
    base_g = wid * N_GROUPS

    pltpu.sync_copy(idx_hbm.at[pl.ds(base_w, B_PER_W)], idx_all)

    lane = lax.iota(jnp.int32, 16)
    row_base = lane * SEQ

    def gather(g, b):
      src = table_hbm.at[idx_all.at[pl.ds(pl.multiple_of(g * CHUNK, 8), CHUNK)]]
      return pltpu.make_async_copy(src, rows[b], sem_g[b])

    def store(g, b):
      gg = base_g + g
      bt = gg // 8
      bi0 = (gg % 8) * BGRP
      dst = out_hbm.at[:, :, bt, :, pl.ds(bi0, BGRP)]
      return pltpu.make_async_copy(trs[b], dst, sem_o[b])

    def transpose(b):
      r = rows[b]
      t = trs[b]

      def srow(s, carry):
        ridx = row_base + s
        for f in range(D):
          v = plsc.load_gather(r, [ridx, jnp.full((16,), f, jnp.int32)])
          t[s, f // 8, f % 8, :] = v
        return carry

      lax.fori_loop(0, SEQ, srow, 0)

    for b in range(NBUF):
      gather(b, b).start()

    for b in range(NBUF):
      gather(b, b).wait()
      transpose(b)
      store(b, b).start()
      gather(b + NBUF, b).start()

    def body(j, carry):
      for b in range(NBUF):
        g = j * NBUF + b
        gather(g, b).wait()
        store(g - NBUF, b).wait()
        transpose(b)
        store(g, b).start()
        gather(g + NBUF, b).start()
      return carry

    lax.fori_loop(1, N_GROUPS // NBUF - 1, body, 0)

    jl = N_GROUPS // NBUF - 1
    for b in range(NBUF):
      g = jl * NBUF + b
      gather(g, b).wait()
      store(g - NBUF, b).wait()
      transpose(b)
      store(g, b).start()
    for b in range(NBUF):
      store(jl * NBUF + b, b).wait()

  return gather_kernel


_gather = _make_kernel()


@jax.jit
def kernel(weights, indices):
  idx_flat = indices.reshape(-1).astype(jnp.int32)
  out6 = _gather(idx_flat, weights)
  t = lax.transpose(out6, (2, 4, 0, 1, 3))
  return t.reshape(NB, SEQ, D)


# speedup vs baseline: 1.2891x; 1.2891x over previous
import functools

import jax
import jax.numpy as jnp
from jax import lax
from jax.experimental import pallas as pl
from jax.experimental.pallas import tpu as pltpu
from jax.experimental.pallas import tpu_sc as plsc

NC = 2
NS = 16
NW = NC * NS

D = 32
NB = 16384
SEQ = 50
B_TOTAL = NB * SEQ
B_PER_W = B_TOTAL // NW
NB_PER_W = NB // NW
BGRP = 16
CHUNK = BGRP * SEQ
N_GROUPS = NB_PER_W // BGRP
NBUF = 2


def _make_kernel():
  mesh = plsc.VectorSubcoreMesh(core_axis_name="c", subcore_axis_name="s")

  @functools.partial(
      pl.kernel,
      mesh=mesh,
      out_type=jax.ShapeDtypeStruct((SEQ, D // 8, NB // 128, 8, 128),
                                    jnp.float32),
      scratch_types=[
          pltpu.VMEM((B_PER_W,), jnp.int32),
          *[pltpu.VMEM((CHUNK, D), jnp.float32) for _ in range(NBUF)],
          *[pltpu.VMEM((SEQ, D // 8, 8, BGRP), jnp.float32)
            for _ in range(NBUF)],
          *[pltpu.SemaphoreType.DMA for _ in range(2 * NBUF)],
      ],
      compiler_params=pltpu.CompilerParams(
          use_tc_tiling_on_sc=False, needs_layout_passes=False
      ),
  )
  def gather_kernel(idx_hbm, table_hbm, out_hbm, idx_all, *bufs_and_sems):
    rows = bufs_and_sems[:NBUF]
    trs = bufs_and_sems[NBUF:2 * NBUF]
    sem_g = bufs_and_sems[2 * NBUF:3 * NBUF]
    sem_o = bufs_and_sems[3 * NBUF:]

    wid = lax.axis_index("s") * NC + lax.axis_index("c")
    base_w = wid * B_PER_W
    base_g = wid * N_GROUPS

    pltpu.sync_copy(idx_hbm.at[pl.ds(base_w, B_PER_W)], idx_all)

    lane = lax.iota(jnp.int32, 16)
    row_base = lane * SEQ

    def gather(g, b):
      src = table_hbm.at[idx_all.at[pl.ds(pl.multiple_of(g * CHUNK, 8), CHUNK)]]
      return pltpu.make_async_copy(src, rows[b], sem_g[b])

    def store(g, b):
      gg = base_g + g
      bt = gg // 8
      bi0 = (gg % 8) * BGRP
      dst = out_hbm.at[:, :, bt, :, pl.ds(bi0, BGRP)]
      return pltpu.make_async_copy(trs[b], dst, sem_o[b])

    def transpose(b):
      r = rows[b]
      t = trs[b]

      def srow(s, carry):
        ridx = row_base + s
        vs = [
            plsc.load_gather(r, [ridx, jnp.full((16,), f, jnp.int32)])
            for f in range(D)
        ]
        for f in range(D):
          t[s, f // 8, f % 8, :] = vs[f]
        return carry

      lax.fori_loop(0, SEQ, srow, 0)

    for b in range(NBUF):
      gather(b, b).start()

    for b in range(NBUF):
      gather(b, b).wait()
      transpose(b)
      store(b, b).start()
      gather(b + NBUF, b).start()

    def body(j, carry):
      for b in range(NBUF):
        g = j * NBUF + b
        gather(g, b).wait()
        store(g - NBUF, b).wait()
        transpose(b)
        store(g, b).start()
        gather(g + NBUF, b).start()
      return carry

    lax.fori_loop(1, N_GROUPS // NBUF - 1, body, 0)

    jl = N_GROUPS // NBUF - 1
    for b in range(NBUF):
      g = jl * NBUF + b
      gather(g, b).wait()
      store(g - NBUF, b).wait()
      transpose(b)
      store(g, b).start()
    for b in range(NBUF):
      store(jl * NBUF + b, b).wait()

  return gather_kernel


_gather = _make_kernel()


@jax.jit
def kernel(weights, indices):
  idx_flat = indices.reshape(-1).astype(jnp.int32)
  out6 = _gather(idx_flat, weights)
  t = lax.transpose(out6, (2, 4, 0, 1, 3))
  return t.reshape(NB, SEQ, D)


# speedup vs baseline: 1.3145x; 1.0197x over previous
import functools

import jax
import jax.numpy as jnp
from jax import lax
from jax.experimental import pallas as pl
from jax.experimental.pallas import tpu as pltpu
from jax.experimental.pallas import tpu_sc as plsc

NC = 2
NS = 16
NW = NC * NS

D = 32
NB = 16384
SEQ = 50
B_TOTAL = NB * SEQ
B_PER_W = B_TOTAL // NW
NB_PER_W = NB // NW
BGRP = 32
HALF = BGRP // 2
CHUNK_H = HALF * SEQ
N_GROUPS = NB_PER_W // BGRP


def _make_kernel():
  mesh = plsc.VectorSubcoreMesh(core_axis_name="c", subcore_axis_name="s")

  @functools.partial(
      pl.kernel,
      mesh=mesh,
      out_type=jax.ShapeDtypeStruct((SEQ, D // 8, NB // 128, 8, 128),
                                    jnp.float32),
      scratch_types=[
          pltpu.VMEM((B_PER_W,), jnp.int32),
          pltpu.VMEM((CHUNK_H, D), jnp.float32),
          pltpu.VMEM((CHUNK_H, D), jnp.float32),
          pltpu.VMEM((SEQ, D // 8, 8, BGRP), jnp.float32),
          pltpu.SemaphoreType.DMA,
          pltpu.SemaphoreType.DMA,
          pltpu.SemaphoreType.DMA,
      ],
      compiler_params=pltpu.CompilerParams(
          use_tc_tiling_on_sc=False, needs_layout_passes=False
      ),
  )
  def gather_kernel(idx_hbm, table_hbm, out_hbm, idx_all, rows0, rows1, trs,
                    sem_g0, sem_g1, sem_o):
    rows = (rows0, rows1)
    sem_g = (sem_g0, sem_g1)

    wid = lax.axis_index("s") * NC + lax.axis_index("c")
    base_w = wid * B_PER_W
    base_g = wid * N_GROUPS

    pltpu.sync_copy(idx_hbm.at[pl.ds(base_w, B_PER_W)], idx_all)

    lane = lax.iota(jnp.int32, 16)
    row_base = lane * SEQ

    def gather(g, h):
      off = pl.multiple_of(g * (2 * CHUNK_H) + h * CHUNK_H, 8)
      src = table_hbm.at[idx_all.at[pl.ds(off, CHUNK_H)]]
      return pltpu.make_async_copy(src, rows[h], sem_g[h])

    def store(g):
      gg = base_g + g
      bt = gg // 4
      bi0 = (gg % 4) * BGRP
      dst = out_hbm.at[:, :, bt, :, pl.ds(bi0, BGRP)]
      return pltpu.make_async_copy(trs, dst, sem_o)

    def transpose(h):
      r = rows[h]

      def srow(s, carry):
        ridx = row_base + s
        vs = [
            plsc.load_gather(r, [ridx, jnp.full((16,), f, jnp.int32)])
            for f in range(D)
        ]
        for f in range(D):
          trs[s, f // 8, f % 8, pl.ds(h * HALF, HALF)] = vs[f]
        return carry

      lax.fori_loop(0, SEQ, srow, 0)

    gather(0, 0).start()
    gather(0, 1).start()
    gather(0, 0).wait()
    transpose(0)
    gather(1, 0).start()
    gather(0, 1).wait()
    transpose(1)
    store(0).start()
    gather(1, 1).start()

    def body(g, carry):
      gather(g, 0).wait()
      store(g - 1).wait()
      transpose(0)
      gather(g + 1, 0).start()
      gather(g, 1).wait()
      transpose(1)
      store(g).start()
      gather(g + 1, 1).start()
      return carry

    lax.fori_loop(1, N_GROUPS - 1, body, 0)

    gl = N_GROUPS - 1
    gather(gl, 0).wait()
    store(gl - 1).wait()
    transpose(0)
    gather(gl, 1).wait()
    transpose(1)
    store(gl).start()
    store(gl).wait()

  return gather_kernel


_gather = _make_kernel()


@jax.jit
def kernel(weights, indices):
  idx_flat = indices.reshape(-1).astype(jnp.int32)
  out6 = _gather(idx_flat, weights)
  t = lax.transpose(out6, (2, 4, 0, 1, 3))
  return t.reshape(NB, SEQ, D)
